# baseline (speedup 1.0000x reference)
import functools

import jax
import jax.numpy as jnp
from jax import lax
from jax.experimental import pallas as pl
from jax.experimental.pallas import tpu as pltpu
from jax.experimental.pallas import tpu_sc as plsc

NUM_USERS = 1000000
NUM_MOVIES = 100000
BATCH = 16384
EMB = 64
ROW = 128
NUM_CORES = 2
NUM_SUBCORES = 16
NUM_WORKERS = NUM_CORES * NUM_SUBCORES
BPW = BATCH // NUM_WORKERS
NCHUNK = 4
CHUNK = BPW // NCHUNK

NTILECOL = (NUM_USERS + 127) // 128
MAXTU = 256
HITCAP = 768
STAGE_ROWS = BATCH + 8
DUMP = BATCH
SCAN = 512


def _stage_user_rows(ue_t, user_idx):
    mesh = plsc.VectorSubcoreMesh(core_axis_name="c", subcore_axis_name="s")

    @functools.partial(
        pl.kernel,
        mesh=mesh,
        compiler_params=pltpu.CompilerParams(
            needs_layout_passes=False, use_tc_tiling_on_sc=True),
        out_type=jax.ShapeDtypeStruct((STAGE_ROWS, ROW), jnp.float32),
        scratch_types=[
            pltpu.VMEM((SCAN,), jnp.int32),
            pltpu.VMEM((HITCAP + 32,), jnp.int32),
            pltpu.VMEM((HITCAP + 32,), jnp.int32),
            pltpu.VMEM((MAXTU + 16,), jnp.int32),
            pltpu.VMEM((MAXTU,), jnp.int32),
            pltpu.VMEM((16,), jnp.int32),
            pltpu.VMEM((16,), jnp.int32),
            pltpu.VMEM((EMB, ROW), jnp.float32),
            pltpu.VMEM((HITCAP, ROW), jnp.float32),
            pltpu.VMEM((HITCAP // 128, 128), jnp.int32),
            pltpu.SemaphoreType.DMA,
        ],
    )
    def ka(uet_hbm, uidx_hbm, stage_hbm,
           idx_v, hu, hb, tul, flags, tmpu, tmpb, tile_v, rowbuf, bposb, sem):
        cid = lax.axis_index("c")
        sid = lax.axis_index("s")
        wid = sid * NUM_CORES + cid

        lane = lax.iota(jnp.int32, 16)
        zero16 = lane * 0
        one16 = zero16 + 1

        @pl.loop(0, MAXTU // 16)
        def _(l):
            flags[pl.ds(l * 16, 16)] = zero16

        @pl.loop(0, HITCAP // 128)
        def _(r):
            @pl.loop(0, 8)
            def _(l):
                bposb[r, pl.ds(l * 16, 16)] = zero16 + DUMP

        def extract(vec, pos):
            return jnp.sum(jnp.where(lane == pos, vec, 0))

        def scan_chunk(ch, off):
            pltpu.sync_copy(uidx_hbm.at[pl.ds(ch * SCAN, SCAN)], idx_v)

            def scan_vec(l, off):
                u16 = idx_v[pl.ds(l * 16, 16)]
                tu16 = lax.shift_right_logical(u16, 7)
                m16 = (tu16 & 31) == wid
                mi = m16.astype(jnp.int32)
                cnt = jnp.sum(mi)
                pos16 = off + jnp.cumsum(mi) - mi
                plsc.store_scatter(hu, [pos16], u16, mask=m16)
                bpos16 = ch * SCAN + l * 16 + lane
                plsc.store_scatter(hb, [pos16], bpos16, mask=m16)
                plsc.store_scatter(flags, [lax.shift_right_logical(u16, 12)],
                                   one16, mask=m16)
                return off + cnt

            return lax.fori_loop(0, SCAN // 16, scan_vec, off)

        nh = lax.fori_loop(0, BATCH // SCAN, scan_chunk, jnp.int32(0))

        def compact_vec(l, offt):
            f16 = flags[pl.ds(l * 16, 16)]
            m16 = f16 != 0
            mi = m16.astype(jnp.int32)
            tu16 = (l * 16 + lane) * 32 + wid
            pos16 = offt + jnp.cumsum(mi) - mi
            plsc.store_scatter(tul, [pos16], tu16, mask=m16)
            return offt + jnp.sum(mi)

        ntu = lax.fori_loop(0, MAXTU // 16, compact_vec, jnp.int32(0))

        nvec = lax.shift_right_logical(nh + 15, 4)

        def do_tile(t, hp):
            tu = extract(tul[pl.ds((lax.shift_right_logical(t, 4)) * 16, 16)],
                         t & 15)
            col = pl.multiple_of(tu * 128, 128)
            cps = [
                pltpu.async_copy(
                    uet_hbm.at[pl.ds(tf * 8, 8), pl.ds(col, 128)],
                    tile_v.at[pl.ds(tf * 8, 8), :], sem)
                for tf in range(8)
            ]
            for c in cps:
                c.wait()

            def hit_vec(v, hp):
                hu16 = hu[pl.ds(v * 16, 16)]
                hb16 = hb[pl.ds(v * 16, 16)]
                hm16 = jnp.logical_and(
                    lax.shift_right_logical(hu16, 7) == tu, (v * 16 + lane) < nh)
                mi = hm16.astype(jnp.int32)
                hcnt = jnp.sum(mi)

                @pl.when(hcnt > 0)
                def _():
                    hp16 = hp + jnp.cumsum(mi) - mi
                    u_in16 = hu16 & 127
                    for f in range(EMB):
                        vals = plsc.load_gather(tile_v, [zero16 + f, u_in16])
                        plsc.store_scatter(rowbuf, [hp16, zero16 + f], vals,
                                           mask=hm16)
                    plsc.store_scatter(
                        bposb,
                        [lax.shift_right_logical(hp16, 7), hp16 & 127],
                        hb16, mask=hm16)

                return hp + hcnt

            return lax.fori_loop(0, nvec, hit_vec, hp)

        lax.fori_loop(0, ntu, do_tile, jnp.int32(0))

        for j in range(HITCAP // 128):
            pltpu.sync_copy(rowbuf.at[pl.ds(j * 128, 128), :],
                            stage_hbm.at[bposb.at[j]])

    return ka(ue_t, user_idx)


def _dot_with_movie(stage, mp, user_bias, movie_bias, user_idx, movie_idx):
    mesh = plsc.VectorSubcoreMesh(core_axis_name="c", subcore_axis_name="s")

    @functools.partial(
        pl.kernel,
        mesh=mesh,
        compiler_params=pltpu.CompilerParams(
            needs_layout_passes=False, use_tc_tiling_on_sc=True),
        out_type=jax.ShapeDtypeStruct((BATCH,), jnp.float32),
        scratch_types=[
            pltpu.VMEM((NCHUNK, CHUNK), jnp.int32),
            pltpu.VMEM((NCHUNK, CHUNK), jnp.int32),
            pltpu.VMEM((CHUNK, ROW), jnp.float32),
            pltpu.VMEM((CHUNK, ROW), jnp.float32),
            pltpu.VMEM((BPW,), jnp.float32),
            pltpu.VMEM((BPW,), jnp.float32),
            pltpu.VMEM((BPW,), jnp.float32),
            pltpu.VMEM((16, 16), jnp.float32),
            pltpu.SemaphoreType.DMA,
            pltpu.SemaphoreType.DMA,
        ],
    )
    def kb(stage_hbm, mp_hbm, ubias_hbm, mbias_hbm, uidx_hbm, midx_hbm, out_hbm,
           midx_v, uidx_v, urows, mrows, ub_v, mb_v, out_v, tr_v, sem, bsem):
        cid = lax.axis_index("c")
        sid = lax.axis_index("s")
        wid = sid * NUM_CORES + cid
        base = wid * BPW

        for j in range(NCHUNK):
            pltpu.sync_copy(midx_hbm.at[pl.ds(base + j * CHUNK, CHUNK)], midx_v.at[j])
            pltpu.sync_copy(uidx_hbm.at[pl.ds(base + j * CHUNK, CHUNK)], uidx_v.at[j])

        for j in range(NCHUNK):
            b1 = pltpu.async_copy(ubias_hbm.at[uidx_v.at[j]], ub_v.at[pl.ds(j * CHUNK, CHUNK)], bsem)
            b2 = pltpu.async_copy(mbias_hbm.at[midx_v.at[j]], mb_v.at[pl.ds(j * CHUNK, CHUNK)], bsem)
            b1.wait()
            b2.wait()

        lane = lax.iota(jnp.int32, 16)
        col15 = lane * 0 + 15

        @pl.loop(0, NCHUNK)
        def _(j):
            g1 = pltpu.async_copy(
                stage_hbm.at[pl.ds(base + j * CHUNK, CHUNK), :], urows, sem)
            g2 = pltpu.async_copy(mp_hbm.at[midx_v.at[j]], mrows, sem)
            g1.wait()
            g2.wait()

            @pl.loop(0, CHUNK // 16)
            def _(g):
                b0 = g * 16
                for i in range(16):
                    b = b0 + i
                    acc = urows[b, pl.ds(0, 16)] * mrows[b, pl.ds(0, 16)]
                    for c in range(1, 4):
                        acc = acc + (urows[b, pl.ds(c * 16, 16)]
                                     * mrows[b, pl.ds(c * 16, 16)])
                    tr_v[i, :] = jnp.cumsum(acc)
                hsum = plsc.load_gather(tr_v, [lane, col15])
                o0 = j * CHUNK + b0
                res = hsum + ub_v[pl.ds(o0, 16)] + mb_v[pl.ds(o0, 16)] + 3.5
                out_v[pl.ds(o0, 16)] = res

        pltpu.sync_copy(out_v, out_hbm.at[pl.ds(base, BPW)])

    return kb(stage, mp, user_bias, movie_bias, user_idx, movie_idx)


def kernel(user_idx, movie_idx, user_embedding, movie_embedding, user_bias, movie_bias):
    uidx = user_idx.astype(jnp.int32)
    midx = movie_idx.astype(jnp.int32)
    stage = _stage_user_rows(user_embedding.T, uidx)
    return _dot_with_movie(
        stage,
        jnp.pad(movie_embedding, ((0, 0), (0, ROW - EMB))),
        user_bias.reshape(-1),
        movie_bias.reshape(-1),
        uidx,
        midx,
    )


# speedup vs baseline: 1.1684x; 1.1684x over previous
import functools

import jax
import jax.numpy as jnp
from jax import lax
from jax.experimental import pallas as pl
from jax.experimental.pallas import tpu as pltpu
from jax.experimental.pallas import tpu_sc as plsc

NUM_USERS = 1000000
NUM_MOVIES = 100000
BATCH = 16384
EMB = 64
ROW = 128
NUM_CORES = 2
NUM_SUBCORES = 16
NUM_WORKERS = NUM_CORES * NUM_SUBCORES
BPW = BATCH // NUM_WORKERS
NCHUNK = 4
CHUNK = BPW // NCHUNK

NTILECOL = (NUM_USERS + 127) // 128
MAXTU = 256
HITCAP = 768
STAGE_ROWS = BATCH + 8
DUMP = BATCH
SCAN = 512


def _stage_user_rows(ue_t, user_idx):
    mesh = plsc.VectorSubcoreMesh(core_axis_name="c", subcore_axis_name="s")

    @functools.partial(
        pl.kernel,
        mesh=mesh,
        compiler_params=pltpu.CompilerParams(
            needs_layout_passes=False, use_tc_tiling_on_sc=True),
        out_type=jax.ShapeDtypeStruct((STAGE_ROWS, ROW), jnp.float32),
        scratch_types=[
            pltpu.VMEM((SCAN,), jnp.int32),
            pltpu.VMEM((HITCAP + 32,), jnp.int32),
            pltpu.VMEM((HITCAP + 32,), jnp.int32),
            pltpu.VMEM((MAXTU + 16,), jnp.int32),
            pltpu.VMEM((MAXTU,), jnp.int32),
            pltpu.VMEM((EMB, ROW), jnp.float32),
            pltpu.VMEM((EMB, ROW), jnp.float32),
            pltpu.VMEM((HITCAP, ROW), jnp.float32),
            pltpu.VMEM((HITCAP // 128, 128), jnp.int32),
            pltpu.SemaphoreType.DMA,
            pltpu.SemaphoreType.DMA,
        ],
    )
    def ka(uet_hbm, uidx_hbm, stage_hbm,
           idx_v, hu, hb, tul, flags, tile_a, tile_b, rowbuf, bposb, sema, semb):
        cid = lax.axis_index("c")
        sid = lax.axis_index("s")
        wid = sid * NUM_CORES + cid

        lane = lax.iota(jnp.int32, 16)
        zero16 = lane * 0
        one16 = zero16 + 1

        @pl.loop(0, MAXTU // 16)
        def _(l):
            flags[pl.ds(l * 16, 16)] = zero16

        @pl.loop(0, HITCAP // 128)
        def _(r):
            @pl.loop(0, 8)
            def _(l):
                bposb[r, pl.ds(l * 16, 16)] = zero16 + DUMP

        def extract(vec, pos):
            return jnp.sum(jnp.where(lane == pos, vec, 0))

        def scan_chunk(ch, off):
            pltpu.sync_copy(uidx_hbm.at[pl.ds(ch * SCAN, SCAN)], idx_v)

            def scan_vec(l, off):
                u16 = idx_v[pl.ds(l * 16, 16)]
                tu16 = lax.shift_right_logical(u16, 7)
                m16 = (tu16 & 31) == wid
                mi = m16.astype(jnp.int32)
                cnt = jnp.sum(mi)
                pos16 = off + jnp.cumsum(mi) - mi
                plsc.store_scatter(hu, [pos16], u16, mask=m16)
                bpos16 = ch * SCAN + l * 16 + lane
                plsc.store_scatter(hb, [pos16], bpos16, mask=m16)
                plsc.store_scatter(flags, [lax.shift_right_logical(u16, 12)],
                                   one16, mask=m16)
                return off + cnt

            return lax.fori_loop(0, SCAN // 16, scan_vec, off)

        nh = lax.fori_loop(0, BATCH // SCAN, scan_chunk, jnp.int32(0))

        @pl.loop(0, (MAXTU + 16) // 16)
        def _(l):
            tul[pl.ds(l * 16, 16)] = zero16 + 32767

        def compact_vec(l, offt):
            f16 = flags[pl.ds(l * 16, 16)]
            m16 = f16 != 0
            mi = m16.astype(jnp.int32)
            tu16 = (l * 16 + lane) * 32 + wid
            pos16 = offt + jnp.cumsum(mi) - mi
            plsc.store_scatter(tul, [pos16], tu16, mask=m16)
            return offt + jnp.sum(mi)

        ntu = lax.fori_loop(0, MAXTU // 16, compact_vec, jnp.int32(0))

        nvec = lax.shift_right_logical(nh + 15, 4)

        def get_tu(t):
            return extract(
                tul[pl.ds((lax.shift_right_logical(t, 4)) * 16, 16)], t & 15)

        def fetch(tu, buf, sem):
            col = pl.multiple_of(jnp.minimum(tu, NTILECOL - 1) * 128, 128)
            for tf in range(8):
                pltpu.async_copy(
                    uet_hbm.at[pl.ds(tf * 8, 8), pl.ds(col, 128)],
                    buf.at[pl.ds(tf * 8, 8), :], sem)

        def drain(buf, sem):
            pltpu.make_async_copy(
                uet_hbm.at[pl.ds(0, EMB), pl.ds(0, ROW)], buf, sem).wait()

        def process(tu, tile_v, hp):
            def hit_vec(v, hp):
                hu16 = hu[pl.ds(v * 16, 16)]
                hb16 = hb[pl.ds(v * 16, 16)]
                hm16 = jnp.logical_and(
                    lax.shift_right_logical(hu16, 7) == tu, (v * 16 + lane) < nh)
                mi = hm16.astype(jnp.int32)
                hcnt = jnp.sum(mi)

                @pl.when(hcnt > 0)
                def _():
                    hp16 = hp + jnp.cumsum(mi) - mi
                    u_in16 = hu16 & 127
                    for f in range(EMB):
                        vals = plsc.load_gather(tile_v, [zero16 + f, u_in16])
                        plsc.store_scatter(rowbuf, [hp16, zero16 + f], vals,
                                           mask=hm16)
                    plsc.store_scatter(
                        bposb,
                        [lax.shift_right_logical(hp16, 7), hp16 & 127],
                        hb16, mask=hm16)

                return hp + hcnt

            return lax.fori_loop(0, nvec, hit_vec, hp)

        tu0 = get_tu(0)
        fetch(tu0, tile_a, sema)

        def pair(g, carry):
            hp, tua = carry
            tub = get_tu(2 * g + 1)
            fetch(tub, tile_b, semb)
            drain(tile_a, sema)
            hp = process(tua, tile_a, hp)
            tua2 = get_tu(2 * g + 2)
            fetch(tua2, tile_a, sema)
            drain(tile_b, semb)
            hp = process(tub, tile_b, hp)
            return hp, tua2

        lax.fori_loop(0, MAXTU // 2, pair, (jnp.int32(0), tu0))
        drain(tile_a, sema)

        for j in range(HITCAP // 128):
            pltpu.sync_copy(rowbuf.at[pl.ds(j * 128, 128), :],
                            stage_hbm.at[bposb.at[j]])

    return ka(ue_t, user_idx)


def _dot_with_movie(stage, mp, user_bias, movie_bias, user_idx, movie_idx):
    mesh = plsc.VectorSubcoreMesh(core_axis_name="c", subcore_axis_name="s")

    @functools.partial(
        pl.kernel,
        mesh=mesh,
        compiler_params=pltpu.CompilerParams(
            needs_layout_passes=False, use_tc_tiling_on_sc=True),
        out_type=jax.ShapeDtypeStruct((BATCH,), jnp.float32),
        scratch_types=[
            pltpu.VMEM((NCHUNK, CHUNK), jnp.int32),
            pltpu.VMEM((NCHUNK, CHUNK), jnp.int32),
            pltpu.VMEM((CHUNK, ROW), jnp.float32),
            pltpu.VMEM((CHUNK, ROW), jnp.float32),
            pltpu.VMEM((BPW,), jnp.float32),
            pltpu.VMEM((BPW,), jnp.float32),
            pltpu.VMEM((BPW,), jnp.float32),
            pltpu.VMEM((16, 16), jnp.float32),
            pltpu.SemaphoreType.DMA,
            pltpu.SemaphoreType.DMA,
        ],
    )
    def kb(stage_hbm, mp_hbm, ubias_hbm, mbias_hbm, uidx_hbm, midx_hbm, out_hbm,
           midx_v, uidx_v, urows, mrows, ub_v, mb_v, out_v, tr_v, sem, bsem):
        cid = lax.axis_index("c")
        sid = lax.axis_index("s")
        wid = sid * NUM_CORES + cid
        base = wid * BPW

        for j in range(NCHUNK):
            pltpu.sync_copy(midx_hbm.at[pl.ds(base + j * CHUNK, CHUNK)], midx_v.at[j])
            pltpu.sync_copy(uidx_hbm.at[pl.ds(base + j * CHUNK, CHUNK)], uidx_v.at[j])

        for j in range(NCHUNK):
            b1 = pltpu.async_copy(ubias_hbm.at[uidx_v.at[j]], ub_v.at[pl.ds(j * CHUNK, CHUNK)], bsem)
            b2 = pltpu.async_copy(mbias_hbm.at[midx_v.at[j]], mb_v.at[pl.ds(j * CHUNK, CHUNK)], bsem)
            b1.wait()
            b2.wait()

        lane = lax.iota(jnp.int32, 16)
        col15 = lane * 0 + 15

        @pl.loop(0, NCHUNK)
        def _(j):
            g1 = pltpu.async_copy(
                stage_hbm.at[pl.ds(base + j * CHUNK, CHUNK), :], urows, sem)
            g2 = pltpu.async_copy(mp_hbm.at[midx_v.at[j]], mrows, sem)
            g1.wait()
            g2.wait()

            @pl.loop(0, CHUNK // 16)
            def _(g):
                b0 = g * 16
                for i in range(16):
                    b = b0 + i
                    acc = urows[b, pl.ds(0, 16)] * mrows[b, pl.ds(0, 16)]
                    for c in range(1, 4):
                        acc = acc + (urows[b, pl.ds(c * 16, 16)]
                                     * mrows[b, pl.ds(c * 16, 16)])
                    tr_v[i, :] = jnp.cumsum(acc)
                hsum = plsc.load_gather(tr_v, [lane, col15])
                o0 = j * CHUNK + b0
                res = hsum + ub_v[pl.ds(o0, 16)] + mb_v[pl.ds(o0, 16)] + 3.5
                out_v[pl.ds(o0, 16)] = res

        pltpu.sync_copy(out_v, out_hbm.at[pl.ds(base, BPW)])

    return kb(stage, mp, user_bias, movie_bias, user_idx, movie_idx)


def kernel(user_idx, movie_idx, user_embedding, movie_embedding, user_bias, movie_bias):
    uidx = user_idx.astype(jnp.int32)
    midx = movie_idx.astype(jnp.int32)
    stage = _stage_user_rows(user_embedding.T, uidx)
    return _dot_with_movie(
        stage,
        jnp.pad(movie_embedding, ((0, 0), (0, ROW - EMB))),
        user_bias.reshape(-1),
        movie_bias.reshape(-1),
        uidx,
        midx,
    )


# speedup vs baseline: 1.2012x; 1.0281x over previous
import functools

import jax
import jax.numpy as jnp
from jax import lax
from jax.experimental import pallas as pl
from jax.experimental.pallas import tpu as pltpu
from jax.experimental.pallas import tpu_sc as plsc

NUM_USERS = 1000000
NUM_MOVIES = 100000
BATCH = 16384
EMB = 64
ROW = 128
NUM_CORES = 2
NUM_SUBCORES = 16
NUM_WORKERS = NUM_CORES * NUM_SUBCORES
BPW = BATCH // NUM_WORKERS
NCHUNK = 4
CHUNK = BPW // NCHUNK

NTILECOL = (NUM_USERS + 127) // 128
MAXTU = 256
HITCAP = 768
STAGE_ROWS = BATCH + 8
DUMP = BATCH
SCAN = 512


def _stage_user_rows(ue_t, user_idx):
    mesh = plsc.VectorSubcoreMesh(core_axis_name="c", subcore_axis_name="s")

    @functools.partial(
        pl.kernel,
        mesh=mesh,
        compiler_params=pltpu.CompilerParams(
            needs_layout_passes=False, use_tc_tiling_on_sc=True),
        out_type=jax.ShapeDtypeStruct((STAGE_ROWS, ROW), jnp.float32),
        scratch_types=[
            pltpu.VMEM((SCAN,), jnp.int32),
            pltpu.VMEM((HITCAP + 32,), jnp.int32),
            pltpu.VMEM((HITCAP + 32,), jnp.int32),
            pltpu.VMEM((MAXTU + 16,), jnp.int32),
            pltpu.VMEM((MAXTU,), jnp.int32),
            pltpu.VMEM((EMB, ROW), jnp.float32),
            pltpu.VMEM((EMB, ROW), jnp.float32),
            pltpu.VMEM((HITCAP, ROW), jnp.float32),
            pltpu.VMEM((HITCAP // 128, 128), jnp.int32),
            pltpu.SemaphoreType.DMA,
            pltpu.SemaphoreType.DMA,
        ],
    )
    def ka(uet_hbm, uidx_hbm, stage_hbm,
           idx_v, hu, hb, tul, flags, tile_a, tile_b, rowbuf, bposb, sema, semb):
        cid = lax.axis_index("c")
        sid = lax.axis_index("s")
        wid = sid * NUM_CORES + cid

        lane = lax.iota(jnp.int32, 16)
        zero16 = lane * 0
        one16 = zero16 + 1

        @pl.loop(0, MAXTU // 16)
        def _(l):
            flags[pl.ds(l * 16, 16)] = zero16

        @pl.loop(0, HITCAP // 128)
        def _(r):
            @pl.loop(0, 8)
            def _(l):
                bposb[r, pl.ds(l * 16, 16)] = zero16 + DUMP

        def extract(vec, pos):
            return jnp.sum(jnp.where(lane == pos, vec, 0))

        def scan_chunk(ch, off):
            pltpu.sync_copy(uidx_hbm.at[pl.ds(ch * SCAN, SCAN)], idx_v)

            def scan_vec(l, off):
                u16 = idx_v[pl.ds(l * 16, 16)]
                tu16 = lax.shift_right_logical(u16, 7)
                m16 = (tu16 & 31) == wid
                mi = m16.astype(jnp.int32)
                cnt = jnp.sum(mi)
                pos16 = off + jnp.cumsum(mi) - mi
                plsc.store_scatter(hu, [pos16], u16, mask=m16)
                bpos16 = ch * SCAN + l * 16 + lane
                plsc.store_scatter(hb, [pos16], bpos16, mask=m16)
                plsc.store_scatter(flags, [lax.shift_right_logical(u16, 12)],
                                   one16, mask=m16)
                return off + cnt

            return lax.fori_loop(0, SCAN // 16, scan_vec, off)

        nh = lax.fori_loop(0, BATCH // SCAN, scan_chunk, jnp.int32(0))

        @pl.loop(0, (MAXTU + 16) // 16)
        def _(l):
            tul[pl.ds(l * 16, 16)] = zero16 + 32767

        def compact_vec(l, offt):
            f16 = flags[pl.ds(l * 16, 16)]
            m16 = f16 != 0
            mi = m16.astype(jnp.int32)
            tu16 = (l * 16 + lane) * 32 + wid
            pos16 = offt + jnp.cumsum(mi) - mi
            plsc.store_scatter(tul, [pos16], tu16, mask=m16)
            return offt + jnp.sum(mi)

        ntu = lax.fori_loop(0, MAXTU // 16, compact_vec, jnp.int32(0))

        nvec = lax.shift_right_logical(nh + 15, 4)

        def get_tu(t):
            return extract(
                tul[pl.ds((lax.shift_right_logical(t, 4)) * 16, 16)], t & 15)

        def fetch(tu, buf, sem):
            col = pl.multiple_of(jnp.minimum(tu, NTILECOL - 1) * 128, 128)
            for tf in range(8):
                pltpu.async_copy(
                    uet_hbm.at[pl.ds(tf * 8, 8), pl.ds(col, 128)],
                    buf.at[pl.ds(tf * 8, 8), :], sem)

        def drain(buf, sem):
            pltpu.make_async_copy(
                uet_hbm.at[pl.ds(0, EMB), pl.ds(0, ROW)], buf, sem).wait()

        def process(tu, tile_v, hp):
            def hit_vec(v, hp):
                hu16 = hu[pl.ds(v * 16, 16)]
                hb16 = hb[pl.ds(v * 16, 16)]
                hm16 = jnp.logical_and(
                    lax.shift_right_logical(hu16, 7) == tu, (v * 16 + lane) < nh)
                mi = hm16.astype(jnp.int32)
                hcnt = jnp.sum(mi)

                @pl.when(hcnt > 0)
                def _():
                    hp16 = hp + jnp.cumsum(mi) - mi
                    u_in16 = hu16 & 127
                    for f in range(EMB):
                        vals = plsc.load_gather(tile_v, [zero16 + f, u_in16])
                        plsc.store_scatter(rowbuf, [hp16, zero16 + f], vals,
                                           mask=hm16)
                    plsc.store_scatter(
                        bposb,
                        [lax.shift_right_logical(hp16, 7), hp16 & 127],
                        hb16, mask=hm16)

                return hp + hcnt

            def hit_vec4(q, hp):
                for k in range(4):
                    hp = hit_vec(4 * q + k, hp)
                return hp

            return lax.fori_loop(0, lax.shift_right_logical(nvec + 3, 2),
                                 hit_vec4, hp)

        tu0 = get_tu(0)
        fetch(tu0, tile_a, sema)

        def pair(g, carry):
            hp, tua = carry
            tub = get_tu(2 * g + 1)
            fetch(tub, tile_b, semb)
            drain(tile_a, sema)
            hp = process(tua, tile_a, hp)
            tua2 = get_tu(2 * g + 2)
            fetch(tua2, tile_a, sema)
            drain(tile_b, semb)
            hp = process(tub, tile_b, hp)
            return hp, tua2

        npair = lax.shift_right_logical(ntu + 2, 1)
        lax.fori_loop(0, npair, pair, (jnp.int32(0), tu0))
        drain(tile_a, sema)

        for j in range(HITCAP // 128):
            pltpu.sync_copy(rowbuf.at[pl.ds(j * 128, 128), :],
                            stage_hbm.at[bposb.at[j]])

    return ka(ue_t, user_idx)


def _dot_with_movie(stage, mp, user_bias, movie_bias, user_idx, movie_idx):
    mesh = plsc.VectorSubcoreMesh(core_axis_name="c", subcore_axis_name="s")

    @functools.partial(
        pl.kernel,
        mesh=mesh,
        compiler_params=pltpu.CompilerParams(
            needs_layout_passes=False, use_tc_tiling_on_sc=True),
        out_type=jax.ShapeDtypeStruct((BATCH,), jnp.float32),
        scratch_types=[
            pltpu.VMEM((NCHUNK, CHUNK), jnp.int32),
            pltpu.VMEM((NCHUNK, CHUNK), jnp.int32),
            pltpu.VMEM((CHUNK, ROW), jnp.float32),
            pltpu.VMEM((CHUNK, ROW), jnp.float32),
            pltpu.VMEM((BPW,), jnp.float32),
            pltpu.VMEM((BPW,), jnp.float32),
            pltpu.VMEM((BPW,), jnp.float32),
            pltpu.VMEM((16, 16), jnp.float32),
            pltpu.SemaphoreType.DMA,
            pltpu.SemaphoreType.DMA,
        ],
    )
    def kb(stage_hbm, mp_hbm, ubias_hbm, mbias_hbm, uidx_hbm, midx_hbm, out_hbm,
           midx_v, uidx_v, urows, mrows, ub_v, mb_v, out_v, tr_v, sem, bsem):
        cid = lax.axis_index("c")
        sid = lax.axis_index("s")
        wid = sid * NUM_CORES + cid
        base = wid * BPW

        for j in range(NCHUNK):
            pltpu.sync_copy(midx_hbm.at[pl.ds(base + j * CHUNK, CHUNK)], midx_v.at[j])
            pltpu.sync_copy(uidx_hbm.at[pl.ds(base + j * CHUNK, CHUNK)], uidx_v.at[j])

        for j in range(NCHUNK):
            b1 = pltpu.async_copy(ubias_hbm.at[uidx_v.at[j]], ub_v.at[pl.ds(j * CHUNK, CHUNK)], bsem)
            b2 = pltpu.async_copy(mbias_hbm.at[midx_v.at[j]], mb_v.at[pl.ds(j * CHUNK, CHUNK)], bsem)
            b1.wait()
            b2.wait()

        lane = lax.iota(jnp.int32, 16)
        col15 = lane * 0 + 15

        @pl.loop(0, NCHUNK)
        def _(j):
            g1 = pltpu.async_copy(
                stage_hbm.at[pl.ds(base + j * CHUNK, CHUNK), :], urows, sem)
            g2 = pltpu.async_copy(mp_hbm.at[midx_v.at[j]], mrows, sem)
            g1.wait()
            g2.wait()

            @pl.loop(0, CHUNK // 16)
            def _(g):
                b0 = g * 16
                for i in range(16):
                    b = b0 + i
                    acc = urows[b, pl.ds(0, 16)] * mrows[b, pl.ds(0, 16)]
                    for c in range(1, 4):
                        acc = acc + (urows[b, pl.ds(c * 16, 16)]
                                     * mrows[b, pl.ds(c * 16, 16)])
                    tr_v[i, :] = jnp.cumsum(acc)
                hsum = plsc.load_gather(tr_v, [lane, col15])
                o0 = j * CHUNK + b0
                res = hsum + ub_v[pl.ds(o0, 16)] + mb_v[pl.ds(o0, 16)] + 3.5
                out_v[pl.ds(o0, 16)] = res

        pltpu.sync_copy(out_v, out_hbm.at[pl.ds(base, BPW)])

    return kb(stage, mp, user_bias, movie_bias, user_idx, movie_idx)


def kernel(user_idx, movie_idx, user_embedding, movie_embedding, user_bias, movie_bias):
    uidx = user_idx.astype(jnp.int32)
    midx = movie_idx.astype(jnp.int32)
    stage = _stage_user_rows(user_embedding.T, uidx)
    return _dot_with_movie(
        stage,
        jnp.pad(movie_embedding, ((0, 0), (0, ROW - EMB))),
        user_bias.reshape(-1),
        movie_bias.reshape(-1),
        uidx,
        midx,
    )


# speedup vs baseline: 1.4141x; 1.1772x over previous
import functools

import jax
import jax.numpy as jnp
from jax import lax
from jax.experimental import pallas as pl
from jax.experimental.pallas import tpu as pltpu
from jax.experimental.pallas import tpu_sc as plsc

NUM_USERS = 1000000
NUM_MOVIES = 100000
BATCH = 16384
EMB = 64
ROW = 128
NUM_CORES = 2
NUM_SUBCORES = 16
NUM_WORKERS = NUM_CORES * NUM_SUBCORES
BPW = BATCH // NUM_WORKERS
NCHUNK = 4
CHUNK = BPW // NCHUNK

NTILECOL = (NUM_USERS + 127) // 128
MAXTU = 256
HITCAP = 768
STAGE_ROWS = BATCH + 8
DUMP = BATCH
SCAN = 512


def _stage_user_rows(ue_t, user_idx):
    mesh = plsc.VectorSubcoreMesh(core_axis_name="c", subcore_axis_name="s")

    @functools.partial(
        pl.kernel,
        mesh=mesh,
        compiler_params=pltpu.CompilerParams(
            needs_layout_passes=False, use_tc_tiling_on_sc=True),
        out_type=jax.ShapeDtypeStruct((STAGE_ROWS, ROW), jnp.float32),
        scratch_types=[
            pltpu.VMEM((SCAN,), jnp.int32),
            pltpu.VMEM((HITCAP + 32,), jnp.int32),
            pltpu.VMEM((HITCAP + 32,), jnp.int32),
            pltpu.VMEM((MAXTU + 16,), jnp.int32),
            pltpu.VMEM((MAXTU,), jnp.int32),
            pltpu.VMEM((16,), jnp.int32),
            pltpu.VMEM((EMB, ROW), jnp.float32),
            pltpu.VMEM((EMB, ROW), jnp.float32),
            pltpu.VMEM((HITCAP, ROW), jnp.float32),
            pltpu.VMEM((HITCAP // 128, 128), jnp.int32),
            pltpu.SemaphoreType.DMA,
            pltpu.SemaphoreType.DMA,
        ],
    )
    def ka(uet_hbm, uidx_hbm, stage_hbm,
           idx_v, hu, hb, tul, flags, tmpu, tile_a, tile_b, rowbuf, bposb,
           sema, semb):
        cid = lax.axis_index("c")
        sid = lax.axis_index("s")
        wid = sid * NUM_CORES + cid

        lane = lax.iota(jnp.int32, 16)
        zero16 = lane * 0
        one16 = zero16 + 1

        @pl.loop(0, MAXTU // 16)
        def _(l):
            flags[pl.ds(l * 16, 16)] = zero16

        @pl.loop(0, HITCAP // 128)
        def _(r):
            @pl.loop(0, 8)
            def _(l):
                bposb[r, pl.ds(l * 16, 16)] = zero16 + DUMP

        def extract(vec, pos):
            return jnp.sum(jnp.where(lane == pos, vec, 0))

        def scan_chunk(ch, off):
            pltpu.sync_copy(uidx_hbm.at[pl.ds(ch * SCAN, SCAN)], idx_v)

            def scan_vec(l, off):
                u16 = idx_v[pl.ds(l * 16, 16)]
                tu16 = lax.shift_right_logical(u16, 7)
                m16 = (tu16 & 31) == wid
                mi = m16.astype(jnp.int32)
                cnt = jnp.sum(mi)
                pos16 = off + jnp.cumsum(mi) - mi
                plsc.store_scatter(hu, [pos16], u16, mask=m16)
                bpos16 = ch * SCAN + l * 16 + lane
                plsc.store_scatter(hb, [pos16], bpos16, mask=m16)
                plsc.store_scatter(flags, [lax.shift_right_logical(u16, 12)],
                                   one16, mask=m16)
                return off + cnt

            return lax.fori_loop(0, SCAN // 16, scan_vec, off)

        nh = lax.fori_loop(0, BATCH // SCAN, scan_chunk, jnp.int32(0))

        @pl.loop(0, (MAXTU + 16) // 16)
        def _(l):
            tul[pl.ds(l * 16, 16)] = zero16 + 32767

        def compact_vec(l, offt):
            f16 = flags[pl.ds(l * 16, 16)]
            m16 = f16 != 0
            mi = m16.astype(jnp.int32)
            tu16 = (l * 16 + lane) * 32 + wid
            pos16 = offt + jnp.cumsum(mi) - mi
            plsc.store_scatter(tul, [pos16], tu16, mask=m16)
            return offt + jnp.sum(mi)

        ntu = lax.fori_loop(0, MAXTU // 16, compact_vec, jnp.int32(0))

        nvec = lax.shift_right_logical(nh + 15, 4)

        def get_tu(t):
            return extract(
                tul[pl.ds((lax.shift_right_logical(t, 4)) * 16, 16)], t & 15)

        def fetch(tu, buf, sem):
            col = pl.multiple_of(jnp.minimum(tu, NTILECOL - 1) * 128, 128)
            for tf in range(8):
                pltpu.async_copy(
                    uet_hbm.at[pl.ds(tf * 8, 8), pl.ds(col, 128)],
                    buf.at[pl.ds(tf * 8, 8), :], sem)

        def drain(buf, sem):
            pltpu.make_async_copy(
                uet_hbm.at[pl.ds(0, EMB), pl.ds(0, ROW)], buf, sem).wait()

        def process(tu, tile_v, hp):
            def hit_vec(v, hp):
                hu16 = hu[pl.ds(v * 16, 16)]
                hb16 = hb[pl.ds(v * 16, 16)]
                hm16 = jnp.logical_and(
                    lax.shift_right_logical(hu16, 7) == tu, (v * 16 + lane) < nh)
                mi = hm16.astype(jnp.int32)
                hcnt = jnp.sum(mi)

                @pl.when(hcnt > 0)
                def _():
                    pos16 = jnp.cumsum(mi) - mi
                    hp16 = hp + pos16
                    plsc.store_scatter(tmpu, [pos16], hu16 & 127, mask=hm16)
                    plsc.store_scatter(
                        bposb,
                        [lax.shift_right_logical(hp16, 7), hp16 & 127],
                        hb16, mask=hm16)
                    cu16 = tmpu[...]

                    def do_hit(s, _):
                        u_in = extract(cu16, s)
                        for c in range(4):
                            vals = plsc.load_gather(
                                tile_v, [c * 16 + lane, zero16 + u_in])
                            rowbuf[hp + s, pl.ds(c * 16, 16)] = vals
                        return _

                    lax.fori_loop(0, hcnt, do_hit, jnp.int32(0))

                return hp + hcnt

            def hit_vec4(q, hp):
                for k in range(4):
                    hp = hit_vec(4 * q + k, hp)
                return hp

            return lax.fori_loop(0, lax.shift_right_logical(nvec + 3, 2),
                                 hit_vec4, hp)

        tu0 = get_tu(0)
        fetch(tu0, tile_a, sema)

        def pair(g, carry):
            hp, tua = carry
            tub = get_tu(2 * g + 1)
            fetch(tub, tile_b, semb)
            drain(tile_a, sema)
            hp = process(tua, tile_a, hp)
            tua2 = get_tu(2 * g + 2)
            fetch(tua2, tile_a, sema)
            drain(tile_b, semb)
            hp = process(tub, tile_b, hp)
            return hp, tua2

        npair = lax.shift_right_logical(ntu + 2, 1)
        lax.fori_loop(0, npair, pair, (jnp.int32(0), tu0))
        drain(tile_a, sema)

        for j in range(HITCAP // 128):
            pltpu.sync_copy(rowbuf.at[pl.ds(j * 128, 128), :],
                            stage_hbm.at[bposb.at[j]])

    return ka(ue_t, user_idx)


def _dot_with_movie(stage, mp, user_bias, movie_bias, user_idx, movie_idx):
    mesh = plsc.VectorSubcoreMesh(core_axis_name="c", subcore_axis_name="s")

    @functools.partial(
        pl.kernel,
        mesh=mesh,
        compiler_params=pltpu.CompilerParams(
            needs_layout_passes=False, use_tc_tiling_on_sc=True),
        out_type=jax.ShapeDtypeStruct((BATCH,), jnp.float32),
        scratch_types=[
            pltpu.VMEM((NCHUNK, CHUNK), jnp.int32),
            pltpu.VMEM((NCHUNK, CHUNK), jnp.int32),
            pltpu.VMEM((CHUNK, ROW), jnp.float32),
            pltpu.VMEM((CHUNK, ROW), jnp.float32),
            pltpu.VMEM((BPW,), jnp.float32),
            pltpu.VMEM((BPW,), jnp.float32),
            pltpu.VMEM((BPW,), jnp.float32),
            pltpu.VMEM((16, 16), jnp.float32),
            pltpu.SemaphoreType.DMA,
            pltpu.SemaphoreType.DMA,
        ],
    )
    def kb(stage_hbm, mp_hbm, ubias_hbm, mbias_hbm, uidx_hbm, midx_hbm, out_hbm,
           midx_v, uidx_v, urows, mrows, ub_v, mb_v, out_v, tr_v, sem, bsem):
        cid = lax.axis_index("c")
        sid = lax.axis_index("s")
        wid = sid * NUM_CORES + cid
        base = wid * BPW

        for j in range(NCHUNK):
            pltpu.sync_copy(midx_hbm.at[pl.ds(base + j * CHUNK, CHUNK)], midx_v.at[j])
            pltpu.sync_copy(uidx_hbm.at[pl.ds(base + j * CHUNK, CHUNK)], uidx_v.at[j])

        for j in range(NCHUNK):
            b1 = pltpu.async_copy(ubias_hbm.at[uidx_v.at[j]], ub_v.at[pl.ds(j * CHUNK, CHUNK)], bsem)
            b2 = pltpu.async_copy(mbias_hbm.at[midx_v.at[j]], mb_v.at[pl.ds(j * CHUNK, CHUNK)], bsem)
            b1.wait()
            b2.wait()

        lane = lax.iota(jnp.int32, 16)
        col15 = lane * 0 + 15

        @pl.loop(0, NCHUNK)
        def _(j):
            g1 = pltpu.async_copy(
                stage_hbm.at[pl.ds(base + j * CHUNK, CHUNK), :], urows, sem)
            g2 = pltpu.async_copy(mp_hbm.at[midx_v.at[j]], mrows, sem)
            g1.wait()
            g2.wait()

            @pl.loop(0, CHUNK // 16)
            def _(g):
                b0 = g * 16
                for i in range(16):
                    b = b0 + i
                    acc = urows[b, pl.ds(0, 16)] * mrows[b, pl.ds(0, 16)]
                    for c in range(1, 4):
                        acc = acc + (urows[b, pl.ds(c * 16, 16)]
                                     * mrows[b, pl.ds(c * 16, 16)])
                    tr_v[i, :] = jnp.cumsum(acc)
                hsum = plsc.load_gather(tr_v, [lane, col15])
                o0 = j * CHUNK + b0
                res = hsum + ub_v[pl.ds(o0, 16)] + mb_v[pl.ds(o0, 16)] + 3.5
                out_v[pl.ds(o0, 16)] = res

        pltpu.sync_copy(out_v, out_hbm.at[pl.ds(base, BPW)])

    return kb(stage, mp, user_bias, movie_bias, user_idx, movie_idx)


def kernel(user_idx, movie_idx, user_embedding, movie_embedding, user_bias, movie_bias):
    uidx = user_idx.astype(jnp.int32)
    midx = movie_idx.astype(jnp.int32)
    stage = _stage_user_rows(user_embedding.T, uidx)
    return _dot_with_movie(
        stage,
        jnp.pad(movie_embedding, ((0, 0), (0, ROW - EMB))),
        user_bias.reshape(-1),
        movie_bias.reshape(-1),
        uidx,
        midx,
    )


# speedup vs baseline: 1.4849x; 1.0501x over previous
import functools

import jax
import jax.numpy as jnp
from jax import lax
from jax.experimental import pallas as pl
from jax.experimental.pallas import tpu as pltpu
from jax.experimental.pallas import tpu_sc as plsc

NUM_USERS = 1000000
NUM_MOVIES = 100000
BATCH = 16384
EMB = 64
ROW = 128
NUM_CORES = 2
NUM_SUBCORES = 16
NUM_WORKERS = NUM_CORES * NUM_SUBCORES
BPW = BATCH // NUM_WORKERS
NCHUNK = 4
CHUNK = BPW // NCHUNK


def _recommender_sc(up, mp, user_bias, movie_bias, user_idx, movie_idx):
    mesh = plsc.VectorSubcoreMesh(core_axis_name="c", subcore_axis_name="s")

    @functools.partial(
        pl.kernel,
        mesh=mesh,
        compiler_params=pltpu.CompilerParams(
            needs_layout_passes=False, use_tc_tiling_on_sc=True),
        out_type=jax.ShapeDtypeStruct((BATCH,), jnp.float32),
        scratch_types=[
            pltpu.VMEM((NCHUNK, CHUNK), jnp.int32),
            pltpu.VMEM((NCHUNK, CHUNK), jnp.int32),
            pltpu.VMEM((CHUNK, ROW), jnp.float32),
            pltpu.VMEM((CHUNK, ROW), jnp.float32),
            pltpu.VMEM((BPW,), jnp.float32),
            pltpu.VMEM((BPW,), jnp.float32),
            pltpu.VMEM((BPW,), jnp.float32),
            pltpu.VMEM((16, 16), jnp.float32),
            pltpu.SemaphoreType.DMA,
            pltpu.SemaphoreType.DMA,
        ],
    )
    def k(up_hbm, mp_hbm, ubias_hbm, mbias_hbm, uidx_hbm, midx_hbm, out_hbm,
          uidx_v, midx_v, urows, mrows, ub_v, mb_v, out_v, tr_v, sem, bsem):
        cid = lax.axis_index("c")
        sid = lax.axis_index("s")
        wid = sid * NUM_CORES + cid
        base = wid * BPW

        for j in range(NCHUNK):
            pltpu.sync_copy(uidx_hbm.at[pl.ds(base + j * CHUNK, CHUNK)], uidx_v.at[j])
            pltpu.sync_copy(midx_hbm.at[pl.ds(base + j * CHUNK, CHUNK)], midx_v.at[j])

        for j in range(NCHUNK):
            b1 = pltpu.async_copy(ubias_hbm.at[uidx_v.at[j]], ub_v.at[pl.ds(j * CHUNK, CHUNK)], bsem)
            b2 = pltpu.async_copy(mbias_hbm.at[midx_v.at[j]], mb_v.at[pl.ds(j * CHUNK, CHUNK)], bsem)
            b1.wait()
            b2.wait()

        lane = lax.iota(jnp.int32, 16)
        col15 = lane * 0 + 15

        @pl.loop(0, NCHUNK)
        def _(j):
            g1 = pltpu.async_copy(up_hbm.at[uidx_v.at[j]], urows, sem)
            g2 = pltpu.async_copy(mp_hbm.at[midx_v.at[j]], mrows, sem)
            g1.wait()
            g2.wait()

            @pl.loop(0, CHUNK // 16)
            def _(g):
                b0 = g * 16
                for i in range(16):
                    b = b0 + i
                    acc = urows[b, pl.ds(0, 16)] * mrows[b, pl.ds(0, 16)]
                    for c in range(1, 4):
                        acc = acc + (urows[b, pl.ds(c * 16, 16)]
                                     * mrows[b, pl.ds(c * 16, 16)])
                    tr_v[i, :] = jnp.cumsum(acc)
                hsum = plsc.load_gather(tr_v, [lane, col15])
                o0 = j * CHUNK + b0
                res = hsum + ub_v[pl.ds(o0, 16)] + mb_v[pl.ds(o0, 16)] + 3.5
                out_v[pl.ds(o0, 16)] = res

        pltpu.sync_copy(out_v, out_hbm.at[pl.ds(base, BPW)])

    return k(up, mp, user_bias, movie_bias, user_idx, movie_idx)


def kernel(user_idx, movie_idx, user_embedding, movie_embedding, user_bias, movie_bias):
    pad = ((0, 0), (0, ROW - EMB))
    return _recommender_sc(
        jnp.pad(user_embedding, pad),
        jnp.pad(movie_embedding, pad),
        user_bias.reshape(-1),
        movie_bias.reshape(-1),
        user_idx.astype(jnp.int32),
        movie_idx.astype(jnp.int32),
    )
